# Initial kernel scaffold; baseline (speedup 1.0000x reference)
import jax
import jax.numpy as jnp
from jax.experimental import pallas as pl


def kernel(seq_inputs, item_inputs, table, W1, b1, W2, b2):
    raise NotImplementedError("write your pallas kernel here")


# speedup vs baseline: 2.3275x; 2.3275x over previous
import functools

import jax
import jax.numpy as jnp
from jax import lax
from jax.experimental import pallas as pl
from jax.experimental.pallas import tpu as pltpu
from jax.experimental.pallas import tpu_sc as plsc

VOCAB = 1000000
D = 64
H = 128
B = 16384
L = 50

NC = 2
NS = 16
NW = NC * NS
PER_W = B // NW
CHUNK = 16
N_CHUNK = PER_W // CHUNK
ROWS = CHUNK * L
IDXCOLS = 100
IDXROWS = ROWS // IDXCOLS


def _sc_gather_sum(seq2d, item_flat, table):
    mesh = plsc.VectorSubcoreMesh(core_axis_name="c", subcore_axis_name="s")

    @functools.partial(
        pl.kernel,
        out_type=(
            jax.ShapeDtypeStruct((B, D), jnp.float32),
            jax.ShapeDtypeStruct((B, D), jnp.float32),
        ),
        mesh=mesh,
        scratch_types=[
            pltpu.VMEM((IDXROWS, IDXCOLS), jnp.int32),
            pltpu.VMEM((ROWS, D), jnp.float32),
            pltpu.VMEM((CHUNK, D), jnp.float32),
            pltpu.VMEM((CHUNK,), jnp.int32),
            pltpu.VMEM((CHUNK, D), jnp.float32),
            pltpu.SemaphoreType.DMA,
            pltpu.SemaphoreType.DMA,
        ],
        compiler_params=pltpu.CompilerParams(use_tc_tiling_on_sc=False),
    )
    def k(seq_hbm, item_hbm, table_hbm, sum_hbm, item_out_hbm,
          idx_v, rows_v, stage_v, iidx_v, irows_v, sem, isem):
        wid = lax.axis_index("s") * NC + lax.axis_index("c")

        def chunk_body(c, carry):
            ebase = pl.multiple_of(wid * PER_W + c * CHUNK, CHUNK)
            irow = pl.multiple_of(ebase * L // IDXCOLS, 8)
            pltpu.sync_copy(seq_hbm.at[pl.ds(irow, IDXROWS)], idx_v)
            pltpu.sync_copy(item_hbm.at[pl.ds(ebase, CHUNK)], iidx_v)
            cps = []
            for j in range(IDXROWS):
                cps.append(pltpu.async_copy(
                    table_hbm.at[idx_v.at[j]],
                    rows_v.at[pl.ds(j * IDXCOLS, IDXCOLS)],
                    sem))
            icp = pltpu.async_copy(table_hbm.at[iidx_v], irows_v, isem)
            for cp in cps:
                cp.wait()
            for b in range(CHUNK):
                def l_body(l, acc):
                    r = b * L + l
                    return tuple(acc[d] + rows_v[r, pl.ds(d * 16, 16)]
                                 for d in range(4))
                acc = lax.fori_loop(
                    0, L, l_body,
                    tuple(jnp.zeros((16,), jnp.float32) for _ in range(4)))
                for d in range(4):
                    stage_v[b, pl.ds(d * 16, 16)] = acc[d]
            icp.wait()
            pltpu.sync_copy(stage_v, sum_hbm.at[pl.ds(ebase, CHUNK)])
            pltpu.sync_copy(irows_v, item_out_hbm.at[pl.ds(ebase, CHUNK)])
            return carry

        lax.fori_loop(0, N_CHUNK, chunk_body, 0)

    return k(seq2d, item_flat, table)


BLK = 512


def _tc_mlp(sums, items, seq, t0, W1, b1, W2, b2):
    def body(sum_ref, item_ref, seq_ref, t0_ref, W1_ref, b1_ref, W2_ref,
             b2_ref, out_ref):
        idx = seq_ref[...]
        n0 = jnp.sum((idx == 0).astype(jnp.float32), axis=1, keepdims=True)
        mean = (sum_ref[...] - n0 * t0_ref[...]) * (1.0 / L)
        h = jnp.maximum(
            jnp.dot(mean, W1_ref[...], preferred_element_type=jnp.float32)
            + b1_ref[...], 0.0)
        u = jnp.maximum(
            jnp.dot(h, W2_ref[...], preferred_element_type=jnp.float32)
            + b2_ref[...], 0.0)
        logit = jnp.sum(u * item_ref[...], axis=1, keepdims=True)
        out_ref[...] = jax.nn.sigmoid(logit)

    return pl.pallas_call(
        body,
        grid=(B // BLK,),
        in_specs=[
            pl.BlockSpec((BLK, D), lambda i: (i, 0)),
            pl.BlockSpec((BLK, D), lambda i: (i, 0)),
            pl.BlockSpec((BLK, L), lambda i: (i, 0)),
            pl.BlockSpec((1, D), lambda i: (0, 0)),
            pl.BlockSpec((D, H), lambda i: (0, 0)),
            pl.BlockSpec((1, H), lambda i: (0, 0)),
            pl.BlockSpec((H, D), lambda i: (0, 0)),
            pl.BlockSpec((1, D), lambda i: (0, 0)),
        ],
        out_specs=pl.BlockSpec((BLK, 1), lambda i: (i, 0)),
        out_shape=jax.ShapeDtypeStruct((B, 1), jnp.float32),
    )(sums, items, seq, t0, W1, b1, W2, b2)


def kernel(seq_inputs, item_inputs, table, W1, b1, W2, b2):
    seq2d = seq_inputs.reshape(B * L // IDXCOLS, IDXCOLS)
    item_flat = item_inputs.reshape(B)
    sums, items = _sc_gather_sum(seq2d, item_flat, table)
    t0 = table[0:1, :]
    return _tc_mlp(sums, items, seq_inputs, t0,
                   W1, b1.reshape(1, H), W2, b2.reshape(1, D))
